# baseline (speedup 1.0000x reference)
import functools

import jax
import jax.numpy as jnp
from jax import lax
from jax.experimental import pallas as pl
from jax.experimental.pallas import tpu as pltpu
from jax.experimental.pallas import tpu_sc as plsc

_B = 16384
_D = 64
_V = 1000000
_NC = 2
_NS = 16
_NW = _NC * _NS
_BPW = _B // _NW
_QPW = _BPW // 2
_CHUNK = 128
_NCH = _BPW // _CHUNK
_L = 16


@functools.partial(
    pl.kernel,
    out_type=jax.ShapeDtypeStruct((_B // 2, 2 * _D), jnp.float32),
    mesh=plsc.VectorSubcoreMesh(core_axis_name="c", subcore_axis_name="s"),
    scratch_types=[
        pltpu.VMEM((_BPW,), jnp.int32),
        pltpu.VMEM((_BPW,), jnp.int32),
        pltpu.VMEM((_BPW, 2 * _D), jnp.float32),
        pltpu.VMEM((_QPW, 2 * _D), jnp.float32),
        pltpu.SemaphoreType.DMA,
    ],
    compiler_params=pltpu.CompilerParams(needs_layout_passes=False),
)
def _embedding_gather2(idx_hbm, tab2_hbm, out2_hbm, idx_v, pair_v, rows_v,
                       sel_v, sem):
    wid = lax.axis_index("s") * _NC + lax.axis_index("c")
    base = wid * _BPW
    pltpu.sync_copy(idx_hbm.at[pl.ds(base, _BPW)], idx_v)

    for v in range(_BPW // _L):
        sl = pl.ds(v * _L, _L)
        pair_v[sl] = lax.shift_right_logical(idx_v[sl], 1)

    copies = [
        pltpu.async_copy(
            tab2_hbm.at[pair_v.at[pl.ds(j * _CHUNK, _CHUNK)]],
            rows_v.at[pl.ds(j * _CHUNK, _CHUNK)],
            sem,
        )
        for j in range(_NCH)
    ]
    for c in copies:
        c.wait()

    lanes = lax.iota(jnp.int32, _L)

    def sel_block(qv, _):
        q0 = qv * _L
        qrow = q0 + lanes
        for h in range(2):
            j_idx = 2 * qrow + h
            ids = plsc.load_gather(idx_v, [j_idx])
            colbase = jnp.bitwise_and(ids, 1) * 64
            for cp in range(_D):
                vals = plsc.load_gather(rows_v, [j_idx, colbase + cp])
                plsc.store_scatter(
                    sel_v, [qrow, jnp.full((_L,), h * _D + cp, jnp.int32)],
                    vals)
        return _

    lax.fori_loop(0, _QPW // _L, sel_block, 0)

    pltpu.sync_copy(sel_v, out2_hbm.at[pl.ds(wid * _QPW, _QPW)])


def kernel(sentence_id, sentence_embedding_weight):
    tab2 = sentence_embedding_weight.reshape(_V // 2, 2 * _D)
    out2 = _embedding_gather2(sentence_id, tab2)
    return out2.reshape(_B, _D)


# speedup vs baseline: 3.7274x; 3.7274x over previous
import functools

import jax
import jax.numpy as jnp
from jax import lax
from jax.experimental import pallas as pl
from jax.experimental.pallas import tpu as pltpu
from jax.experimental.pallas import tpu_sc as plsc

_B = 16384
_D = 64
_V = 1000000
_NC = 2
_NS = 16
_NW = _NC * _NS
_L = 16
_NCOL = (_V + 127) // 128
_CPW = _NCOL // _NW
_CREM = _NCOL - _CPW * _NW
_NVR = _B // _L
_BATCH = 128
_NBUF = 3
_NBKT = 8


def _splat(x):
    return jnp.full((_L,), x, jnp.int32)


@functools.partial(
    pl.kernel,
    out_type=jax.ShapeDtypeStruct((_B, 2 * _D), jnp.float32),
    mesh=plsc.VectorSubcoreMesh(core_axis_name="c", subcore_axis_name="s"),
    scratch_types=[
        pltpu.VMEM((_B,), jnp.int32),
        pltpu.VMEM((_B,), jnp.int32),
        pltpu.VMEM((_B,), jnp.int32),
        pltpu.VMEM((_B,), jnp.int32),
        pltpu.VMEM((_B,), jnp.int32),
        pltpu.VMEM((_L,), jnp.int32),
        pltpu.VMEM((_L,), jnp.int32),
        pltpu.VMEM((_NBUF, _D, 128), jnp.float32),
        pltpu.VMEM((_BATCH, 2 * _D), jnp.float32),
        pltpu.VMEM((1, _BATCH), jnp.int32),
        pltpu.SemaphoreType.DMA,
        pltpu.SemaphoreType.DMA,
        pltpu.SemaphoreType.DMA,
        pltpu.SemaphoreType.DMA,
    ],
    compiler_params=pltpu.CompilerParams(
        needs_layout_passes=False, disable_bounds_checks=True),
)
def _scan_gather(idx_hbm, tabt_hbm, out_hbm, idx_v, ids_list, pos_list,
                 bkt_ids, bkt_pos, bstart, bend, grp_v, rows_acc, j_acc,
                 sem0, sem1, sem2, semw):
    wid = lax.axis_index("s") * _NC + lax.axis_index("c")
    c_lo = _CPW * wid + jnp.minimum(wid, _CREM)
    ncols = _CPW + jnp.where(wid < _CREM, 1, 0)

    pltpu.sync_copy(idx_hbm, idx_v)

    lanes = lax.iota(jnp.int32, _L)
    lane0 = lanes == _splat(0)
    c_lo_v = _splat(c_lo)
    c_hi_v = _splat(c_lo + ncols)

    def scan_body(v, cnt):
        ivec = plsc.load_gather(idx_v, [v * _L + lanes])
        cvec = lax.shift_right_logical(ivec, 7)
        m = jnp.logical_and(cvec >= c_lo_v, cvec < c_hi_v)
        mi = jnp.where(m, 1, 0)
        cum = plsc.cumsum(mi)
        slots = _splat(cnt) + cum - mi
        plsc.store_scatter(ids_list, [slots], ivec, mask=m)
        plsc.store_scatter(pos_list, [slots], v * _L + lanes, mask=m)
        return cnt + jnp.max(cum)

    cnt = lax.fori_loop(0, _NVR, scan_body, jnp.int32(0))
    nv = lax.div(cnt + _L - 1, _L)
    cnt_v = _splat(cnt)

    bptr = jnp.int32(0)
    for k in range(_NBKT):
        plsc.store_scatter(bstart, [_splat(k)], _splat(bptr), mask=lane0)

        def bkt_body(v, bp, k=k):
            vl = v * _L + lanes
            ivec = plsc.load_gather(ids_list, [vl])
            pvec = plsc.load_gather(pos_list, [vl])
            cvec = lax.shift_right_logical(ivec, 7)
            m = jnp.logical_and(
                lax.shift_right_logical(cvec - c_lo_v, 5) == _splat(k),
                vl < cnt_v)
            mi = jnp.where(m, 1, 0)
            cum = plsc.cumsum(mi)
            slots = _splat(bp) + cum - mi
            plsc.store_scatter(bkt_ids, [slots], ivec, mask=m)
            plsc.store_scatter(bkt_pos, [slots], pvec, mask=m)
            return bp + jnp.max(cum)

        bptr = lax.fori_loop(0, nv, bkt_body, bptr)
        plsc.store_scatter(bend, [_splat(k)], _splat(bptr), mask=lane0)

    sems = (sem0, sem1, sem2)

    def issue_dma(c, b, sem):
        pltpu.async_copy(
            tabt_hbm.at[:, pl.ds(pl.multiple_of(c * 128, 128), 128)],
            grp_v.at[b], sem)

    def wait_dma(b, sem):
        pltpu.make_async_copy(
            tabt_hbm.at[:, pl.ds(0, 128)], grp_v.at[b], sem
        ).wait()

    for b in range(_NBUF):
        @pl.when(ncols > b)
        def _prime(b=b):
            issue_dma(c_lo + b, b, sems[b])

    def process_column(ci, carry):
        acc, last_valid = carry
        b = lax.rem(ci, _NBUF)
        c = c_lo + ci

        for bb in range(_NBUF):
            @pl.when(b == bb)
            def _w(bb=bb):
                wait_dma(bb, sems[bb])

        wb_v = _splat(c * 128)
        c_v = _splat(c)
        kc = lax.shift_right_logical(c - c_lo, 5)
        s0 = jnp.max(plsc.load_gather(bstart, [_splat(kc)]))
        e0 = jnp.max(plsc.load_gather(bend, [_splat(kc)]))
        e0_v = _splat(e0)
        nvb = lax.div(e0 - s0 + _L - 1, _L)

        def match_body(v, mc):
            vl = _splat(s0) + v * _L + lanes
            ivec = plsc.load_gather(bkt_ids, [vl])
            pvec = plsc.load_gather(bkt_pos, [vl])
            m = jnp.logical_and(
                lax.shift_right_logical(ivec, 7) == c_v, vl < e0_v)
            mi = jnp.where(m, 1, 0)
            cum = plsc.cumsum(mi)
            slots = _splat(mc) + cum - mi
            plsc.store_scatter(ids_list, [slots], ivec - wb_v, mask=m)
            plsc.store_scatter(pos_list, [slots], pvec, mask=m)
            return mc + jnp.max(cum)

        nm = lax.fori_loop(0, nvb, match_body, jnp.int32(0))

        def extract_body(h, ec):
            acc2, _lv = ec
            lc = plsc.load_gather(ids_list, [_splat(h)])
            jj = plsc.load_gather(pos_list, [_splat(h)])
            slot = lax.rem(acc2, _BATCH)
            for q in range(_D // _L):
                vals = plsc.load_gather(
                    grp_v, [_splat(b), lanes + q * _L, lc])
                plsc.store_scatter(
                    rows_acc, [_splat(slot), lanes + q * _L], vals)
            plsc.store_scatter(j_acc, [_splat(0), _splat(slot)], jj,
                               mask=lane0)

            @pl.when(slot == _BATCH - 1)
            def _flush():
                pltpu.async_copy(
                    rows_acc, out_hbm.at[j_acc.at[0]], semw).wait()

            return acc2 + 1, jnp.max(jj)

        acc, last_valid = lax.fori_loop(
            0, nm, extract_body, (acc, last_valid))

        @pl.when(ci + _NBUF < ncols)
        def _next():
            for bb in range(_NBUF):
                @pl.when(b == bb)
                def _n(bb=bb):
                    issue_dma(c + _NBUF, bb, sems[bb])

        return acc, last_valid

    acc, last_valid = lax.fori_loop(
        0, ncols, process_column, (jnp.int32(0), jnp.int32(0)))

    tail = lax.rem(acc, _BATCH)

    @pl.when(jnp.logical_and(tail > 0, acc > 0))
    def _tail():
        lastslot = tail - 1

        def fill_body(s, _):
            for q in range(_D // _L):
                vals = plsc.load_gather(
                    rows_acc, [_splat(lastslot), lanes + q * _L])
                plsc.store_scatter(
                    rows_acc, [_splat(s), lanes + q * _L], vals)
            plsc.store_scatter(j_acc, [_splat(0), _splat(s)],
                               _splat(last_valid), mask=lane0)
            return _

        lax.fori_loop(tail, _BATCH, fill_body, 0)
        pltpu.async_copy(rows_acc, out_hbm.at[j_acc.at[0]], semw).wait()


def kernel(sentence_id, sentence_embedding_weight):
    inter = _scan_gather(sentence_id, sentence_embedding_weight.T)
    return inter[:, :_D]


# speedup vs baseline: 3.7571x; 1.0080x over previous
import functools

import jax
import jax.numpy as jnp
from jax import lax
from jax.experimental import pallas as pl
from jax.experimental.pallas import tpu as pltpu
from jax.experimental.pallas import tpu_sc as plsc

_B = 16384
_D = 64
_V = 1000000
_NC = 2
_NS = 16
_NW = _NC * _NS
_L = 16
_NCOL = (_V + 127) // 128
_CPW = _NCOL // _NW
_CREM = _NCOL - _CPW * _NW
_NVR = _B // _L
_BATCH = 128
_NBUF = 3
_NBKT = 8


def _splat(x):
    return jnp.full((_L,), x, jnp.int32)


@functools.partial(
    pl.kernel,
    out_type=jax.ShapeDtypeStruct((_B, 2 * _D), jnp.float32),
    mesh=plsc.VectorSubcoreMesh(core_axis_name="c", subcore_axis_name="s"),
    scratch_types=[
        pltpu.VMEM((_B,), jnp.int32),
        pltpu.VMEM((_B,), jnp.int32),
        pltpu.VMEM((_B,), jnp.int32),
        pltpu.VMEM((_B,), jnp.int32),
        pltpu.VMEM((_B,), jnp.int32),
        pltpu.VMEM((_L,), jnp.int32),
        pltpu.VMEM((_L,), jnp.int32),
        pltpu.VMEM((_NBUF, _D, 128), jnp.float32),
        pltpu.VMEM((_BATCH, 2 * _D), jnp.float32),
        pltpu.VMEM((1, _BATCH), jnp.int32),
        pltpu.SemaphoreType.DMA,
        pltpu.SemaphoreType.DMA,
        pltpu.SemaphoreType.DMA,
        pltpu.SemaphoreType.DMA,
    ],
    compiler_params=pltpu.CompilerParams(
        needs_layout_passes=False, disable_bounds_checks=True),
)
def _scan_gather(idx_hbm, tabt_hbm, out_hbm, idx_v, ids_list, pos_list,
                 bkt_ids, bkt_pos, bstart, bend, grp_v, rows_acc, j_acc,
                 sem0, sem1, sem2, semw):
    wid = lax.axis_index("s") * _NC + lax.axis_index("c")
    c_lo = _CPW * wid + jnp.minimum(wid, _CREM)
    ncols = _CPW + jnp.where(wid < _CREM, 1, 0)

    pltpu.sync_copy(idx_hbm, idx_v)

    lanes = lax.iota(jnp.int32, _L)
    lane0 = lanes == _splat(0)
    c_lo_v = _splat(c_lo)
    c_hi_v = _splat(c_lo + ncols)

    sems = (sem0, sem1, sem2)

    def issue_dma(c, b, sem):
        pltpu.async_copy(
            tabt_hbm.at[:, pl.ds(pl.multiple_of(c * 128, 128), 128)],
            grp_v.at[b], sem)

    def wait_dma(b, sem):
        pltpu.make_async_copy(
            tabt_hbm.at[:, pl.ds(0, 128)], grp_v.at[b], sem
        ).wait()

    for b in range(_NBUF):
        @pl.when(ncols > b)
        def _prime(b=b):
            issue_dma(c_lo + b, b, sems[b])

    def scan_body(v, cnt):
        ivec = plsc.load_gather(idx_v, [v * _L + lanes])
        cvec = lax.shift_right_logical(ivec, 7)
        m = jnp.logical_and(cvec >= c_lo_v, cvec < c_hi_v)
        mi = jnp.where(m, 1, 0)
        cum = plsc.cumsum(mi)
        slots = _splat(cnt) + cum - mi
        plsc.store_scatter(ids_list, [slots], ivec, mask=m)
        plsc.store_scatter(pos_list, [slots], v * _L + lanes, mask=m)
        return cnt + jnp.max(cum)

    cnt = lax.fori_loop(0, _NVR, scan_body, jnp.int32(0))
    nv = lax.div(cnt + _L - 1, _L)
    cnt_v = _splat(cnt)

    bptr = jnp.int32(0)
    for k in range(_NBKT):
        plsc.store_scatter(bstart, [_splat(k)], _splat(bptr), mask=lane0)

        def bkt_body(v, bp, k=k):
            vl = v * _L + lanes
            ivec = plsc.load_gather(ids_list, [vl])
            pvec = plsc.load_gather(pos_list, [vl])
            cvec = lax.shift_right_logical(ivec, 7)
            m = jnp.logical_and(
                lax.shift_right_logical(cvec - c_lo_v, 5) == _splat(k),
                vl < cnt_v)
            mi = jnp.where(m, 1, 0)
            cum = plsc.cumsum(mi)
            slots = _splat(bp) + cum - mi
            plsc.store_scatter(bkt_ids, [slots], ivec, mask=m)
            plsc.store_scatter(bkt_pos, [slots], pvec, mask=m)
            return bp + jnp.max(cum)

        bptr = lax.fori_loop(0, nv, bkt_body, bptr)
        plsc.store_scatter(bend, [_splat(k)], _splat(bptr), mask=lane0)

    def process_column(ci, carry):
        acc, last_valid = carry
        b = lax.rem(ci, _NBUF)
        c = c_lo + ci

        for bb in range(_NBUF):
            @pl.when(b == bb)
            def _w(bb=bb):
                wait_dma(bb, sems[bb])

        wb_v = _splat(c * 128)
        c_v = _splat(c)
        kc = lax.shift_right_logical(c - c_lo, 5)
        s0 = jnp.max(plsc.load_gather(bstart, [_splat(kc)]))
        e0 = jnp.max(plsc.load_gather(bend, [_splat(kc)]))
        e0_v = _splat(e0)
        nvb = lax.div(e0 - s0 + _L - 1, _L)

        def match_body(v, mc):
            vl = _splat(s0) + v * _L + lanes
            ivec = plsc.load_gather(bkt_ids, [vl])
            pvec = plsc.load_gather(bkt_pos, [vl])
            m = jnp.logical_and(
                lax.shift_right_logical(ivec, 7) == c_v, vl < e0_v)
            mi = jnp.where(m, 1, 0)
            cum = plsc.cumsum(mi)
            slots = _splat(mc) + cum - mi
            plsc.store_scatter(ids_list, [slots], ivec - wb_v, mask=m)
            plsc.store_scatter(pos_list, [slots], pvec, mask=m)
            return mc + jnp.max(cum)

        nm = lax.fori_loop(0, nvb, match_body, jnp.int32(0))

        def extract_body(h, ec):
            acc2, _lv = ec
            lc = plsc.load_gather(ids_list, [_splat(h)])
            jj = plsc.load_gather(pos_list, [_splat(h)])
            slot = lax.rem(acc2, _BATCH)
            for q in range(_D // _L):
                vals = plsc.load_gather(
                    grp_v, [_splat(b), lanes + q * _L, lc])
                plsc.store_scatter(
                    rows_acc, [_splat(slot), lanes + q * _L], vals)
            plsc.store_scatter(j_acc, [_splat(0), _splat(slot)], jj,
                               mask=lane0)

            @pl.when(slot == _BATCH - 1)
            def _flush():
                pltpu.async_copy(
                    rows_acc, out_hbm.at[j_acc.at[0]], semw).wait()

            return acc2 + 1, jnp.max(jj)

        acc, last_valid = lax.fori_loop(
            0, nm, extract_body, (acc, last_valid))

        @pl.when(ci + _NBUF < ncols)
        def _next():
            for bb in range(_NBUF):
                @pl.when(b == bb)
                def _n(bb=bb):
                    issue_dma(c + _NBUF, bb, sems[bb])

        return acc, last_valid

    acc, last_valid = lax.fori_loop(
        0, ncols, process_column, (jnp.int32(0), jnp.int32(0)))

    tail = lax.rem(acc, _BATCH)

    @pl.when(jnp.logical_and(tail > 0, acc > 0))
    def _tail():
        lastslot = tail - 1

        def fill_body(s, _):
            for q in range(_D // _L):
                vals = plsc.load_gather(
                    rows_acc, [_splat(lastslot), lanes + q * _L])
                plsc.store_scatter(
                    rows_acc, [_splat(s), lanes + q * _L], vals)
            plsc.store_scatter(j_acc, [_splat(0), _splat(s)],
                               _splat(last_valid), mask=lane0)
            return _

        lax.fori_loop(tail, _BATCH, fill_body, 0)
        pltpu.async_copy(rows_acc, out_hbm.at[j_acc.at[0]], semw).wait()


def kernel(sentence_id, sentence_embedding_weight):
    inter = _scan_gather(sentence_id, sentence_embedding_weight.T)
    return inter[:, :_D]
